# baseline (speedup 1.0000x reference)
import functools

import jax
import jax.numpy as jnp
import numpy as np
from jax.experimental import pallas as pl
from jax.experimental.pallas import tpu as pltpu
from jax.sharding import Mesh, PartitionSpec as P


def _mm_stats_kernel(x_ref, e_ref, w_ref, y_ref, sum_ref, sq_ref,
                     xw_ref, *, bm):
    i = pl.program_id(0)

    @pl.when(i == 0)
    def _init():
        xw_ref[...] = jnp.dot(x_ref[...], w_ref[...],
                              preferred_element_type=jnp.float32
                              ).astype(jnp.bfloat16)
        sum_ref[...] = jnp.zeros_like(sum_ref)
        sq_ref[...] = jnp.zeros_like(sq_ref)

    y = jnp.dot(e_ref[...].astype(jnp.bfloat16), xw_ref[...],
                preferred_element_type=jnp.float32)
    y_ref[pl.ds(i * bm, bm), :] = y
    sum_ref[...] += jnp.sum(y, axis=0, keepdims=True)
    sq_ref[...] += jnp.sum(y * y, axis=0, keepdims=True)


def _bn_act_kernel(y_ref, sum_ref, sq_ref, g_ref, b_ref, o_ref, *, n):
    mean = sum_ref[...] * (1.0 / n)
    var = sq_ref[...] * (1.0 / n) - mean * mean
    scale = g_ref[...] * jax.lax.rsqrt(var + 1e-5)
    shift = b_ref[...] - mean * scale
    z = y_ref[...] * scale + shift
    o_ref[...] = jnp.where(z >= 0, z, 0.01 * z)


def _local_pipeline(n, d_in, d_out, rows, bm, X, E_rows, W, gamma, beta):
    nsteps = rows // bm
    y, colsum, colsq = pl.pallas_call(
        functools.partial(_mm_stats_kernel, bm=bm),
        grid=(nsteps,),
        in_specs=[
            pl.BlockSpec((n, d_in), lambda i: (0, 0)),
            pl.BlockSpec((bm, n), lambda i: (i, 0)),
            pl.BlockSpec((d_in, d_out), lambda i: (0, 0)),
        ],
        out_specs=[
            pl.BlockSpec((rows, d_out), lambda i: (0, 0)),
            pl.BlockSpec((1, d_out), lambda i: (0, 0)),
            pl.BlockSpec((1, d_out), lambda i: (0, 0)),
        ],
        out_shape=[
            jax.ShapeDtypeStruct((rows, d_out), jnp.float32),
            jax.ShapeDtypeStruct((1, d_out), jnp.float32),
            jax.ShapeDtypeStruct((1, d_out), jnp.float32),
        ],
        scratch_shapes=[pltpu.VMEM((n, d_out), jnp.bfloat16)],
    )(X, E_rows, W)
    return y, colsum, colsq


def _normalize(n, d_out, rows, y, colsum, colsq, gamma, beta):
    return pl.pallas_call(
        functools.partial(_bn_act_kernel, n=float(n)),
        grid=(1,),
        in_specs=[
            pl.BlockSpec((rows, d_out), lambda i: (0, 0)),
            pl.BlockSpec((1, d_out), lambda i: (0, 0)),
            pl.BlockSpec((1, d_out), lambda i: (0, 0)),
            pl.BlockSpec((1, d_out), lambda i: (0, 0)),
            pl.BlockSpec((1, d_out), lambda i: (0, 0)),
        ],
        out_specs=pl.BlockSpec((rows, d_out), lambda i: (0, 0)),
        out_shape=jax.ShapeDtypeStruct((rows, d_out), jnp.float32),
    )(y, colsum, colsq, gamma, beta)


def kernel(X, E, W, bias, gamma, beta):
    del bias
    n, d_in = X.shape
    d_out = W.shape[1]
    gamma2 = gamma.reshape(1, d_out)
    beta2 = beta.reshape(1, d_out)

    ndev = jax.device_count()
    bm = None
    for cand_ndev in (ndev, 1):
        if n % cand_ndev:
            continue
        rows = n // cand_ndev
        for cand_bm in (400, 200, 1000, 8):
            if rows % cand_bm == 0:
                ndev, bm = cand_ndev, cand_bm
                break
        if bm is not None:
            break
    rows = n // ndev

    if ndev == 1:
        y, colsum, colsq = _local_pipeline(n, d_in, d_out, n, bm,
                                           X, E, W, gamma2, beta2)
        return _normalize(n, d_out, n, y, colsum, colsq, gamma2, beta2)

    mesh = Mesh(np.array(jax.devices()[:ndev]), ("d",))

    @functools.partial(
        jax.shard_map, mesh=mesh, check_vma=False,
        in_specs=(P(), P("d", None), P(), P(), P()),
        out_specs=P("d", None))
    def _sharded(X, E_rows, W, gamma2, beta2):
        y, colsum, colsq = _local_pipeline(n, d_in, d_out, rows, bm,
                                           X, E_rows, W, gamma2, beta2)
        colsum = jax.lax.psum(colsum, "d")
        colsq = jax.lax.psum(colsq, "d")
        return _normalize(n, d_out, rows, y, colsum, colsq, gamma2, beta2)

    return _sharded(X, E, W, gamma2, beta2)


# speedup vs baseline: 5.4766x; 5.4766x over previous
import functools

import jax
import jax.numpy as jnp
from jax.experimental import pallas as pl
from jax.experimental.pallas import tpu as pltpu


def _fused_kernel(x_ref, e_ref, w_ref, g_ref, b_ref, o_ref,
                  y_ref, xw_ref, sum_ref, sq_ref, sc_ref, sh_ref,
                  *, nsteps, bm, n):
    i = pl.program_id(0)

    @pl.when(i == 0)
    def _init():
        xw_ref[...] = jnp.dot(x_ref[...], w_ref[...],
                              preferred_element_type=jnp.float32
                              ).astype(jnp.bfloat16)
        sum_ref[...] = jnp.zeros_like(sum_ref)
        sq_ref[...] = jnp.zeros_like(sq_ref)

    @pl.when(i < nsteps)
    def _matmul_step():
        y = jnp.dot(e_ref[...].astype(jnp.bfloat16), xw_ref[...],
                    preferred_element_type=jnp.float32)
        y_ref[pl.ds(i * bm, bm), :] = y
        sum_ref[...] += jnp.sum(y, axis=0, keepdims=True)
        sq_ref[...] += jnp.sum(y * y, axis=0, keepdims=True)

    @pl.when(i == nsteps - 1)
    def _close_stats():
        mean = sum_ref[...] * (1.0 / n)
        var = sq_ref[...] * (1.0 / n) - mean * mean
        scale = g_ref[...] * jax.lax.rsqrt(var + 1e-5)
        sc_ref[...] = scale
        sh_ref[...] = b_ref[...] - mean * scale

    @pl.when(i >= nsteps)
    def _epilogue_step():
        j = i - nsteps
        z = y_ref[pl.ds(j * bm, bm), :] * sc_ref[...] + sh_ref[...]
        o_ref[...] = jnp.maximum(z, 0.01 * z)


def kernel(X, E, W, bias, gamma, beta):
    del bias
    n, d_in = X.shape
    d_out = W.shape[1]
    bm = 400
    nsteps = n // bm

    return pl.pallas_call(
        functools.partial(_fused_kernel, nsteps=nsteps, bm=bm, n=float(n)),
        grid=(2 * nsteps,),
        in_specs=[
            pl.BlockSpec((n, d_in), lambda i: (0, 0)),
            pl.BlockSpec((bm, n),
                         lambda i, _ns=nsteps: (jnp.minimum(i, _ns - 1), 0)),
            pl.BlockSpec((d_in, d_out), lambda i: (0, 0)),
            pl.BlockSpec((1, d_out), lambda i: (0, 0)),
            pl.BlockSpec((1, d_out), lambda i: (0, 0)),
        ],
        out_specs=pl.BlockSpec(
            (bm, d_out),
            lambda i, _ns=nsteps: (jnp.maximum(i - _ns, 0), 0)),
        out_shape=jax.ShapeDtypeStruct((n, d_out), jnp.float32),
        scratch_shapes=[
            pltpu.VMEM((n, d_out), jnp.float32),
            pltpu.VMEM((n, d_out), jnp.bfloat16),
            pltpu.VMEM((1, d_out), jnp.float32),
            pltpu.VMEM((1, d_out), jnp.float32),
            pltpu.VMEM((1, d_out), jnp.float32),
            pltpu.VMEM((1, d_out), jnp.float32),
        ],
    )(X, E, W, gamma.reshape(1, d_out), beta.reshape(1, d_out))


# speedup vs baseline: 5.7658x; 1.0528x over previous
import functools

import jax
import jax.numpy as jnp
from jax.experimental import pallas as pl
from jax.experimental.pallas import tpu as pltpu


def _fused_kernel(x_ref, e_ref, w_ref, g_ref, b_ref, o_ref,
                  xw_ref, sum_ref, sq_ref, *, nsteps, bm, n):
    i = pl.program_id(0)

    @pl.when(i == 0)
    def _init():
        xw_ref[...] = jnp.dot(x_ref[...], w_ref[...],
                              preferred_element_type=jnp.float32
                              ).astype(jnp.bfloat16)
        sum_ref[...] = jnp.zeros_like(sum_ref)
        sq_ref[...] = jnp.zeros_like(sq_ref)

    y = jnp.dot(e_ref[...].astype(jnp.bfloat16), xw_ref[...],
                preferred_element_type=jnp.float32)
    o_ref[pl.ds(i * bm, bm), :] = y
    sum_ref[...] += jnp.sum(y, axis=0, keepdims=True)
    sq_ref[...] += jnp.sum(y * y, axis=0, keepdims=True)

    @pl.when(i == nsteps - 1)
    def _finalize():
        mean = sum_ref[...] * (1.0 / n)
        var = sq_ref[...] * (1.0 / n) - mean * mean
        scale = g_ref[...] * jax.lax.rsqrt(var + 1e-5)
        shift = b_ref[...] - mean * scale
        z = o_ref[...] * scale + shift
        o_ref[...] = jnp.maximum(z, 0.01 * z)


def kernel(X, E, W, bias, gamma, beta):
    del bias
    n, d_in = X.shape
    d_out = W.shape[1]
    bm = 400
    nsteps = n // bm

    return pl.pallas_call(
        functools.partial(_fused_kernel, nsteps=nsteps, bm=bm, n=float(n)),
        grid=(nsteps,),
        in_specs=[
            pl.BlockSpec((n, d_in), lambda i: (0, 0)),
            pl.BlockSpec((bm, n), lambda i: (i, 0)),
            pl.BlockSpec((d_in, d_out), lambda i: (0, 0)),
            pl.BlockSpec((1, d_out), lambda i: (0, 0)),
            pl.BlockSpec((1, d_out), lambda i: (0, 0)),
        ],
        out_specs=pl.BlockSpec((n, d_out), lambda i: (0, 0)),
        out_shape=jax.ShapeDtypeStruct((n, d_out), jnp.float32),
        scratch_shapes=[
            pltpu.VMEM((n, d_out), jnp.bfloat16),
            pltpu.VMEM((1, d_out), jnp.float32),
            pltpu.VMEM((1, d_out), jnp.float32),
        ],
    )(X, E, W, gamma.reshape(1, d_out), beta.reshape(1, d_out))
